# baseline (speedup 1.0000x reference)
import jax
import jax.numpy as jnp
from jax.experimental import pallas as pl

EPS = 1e-5
B = 8
NC = 40


def _fold(p):
    s = [g * (1.0 / jnp.sqrt(1.0 + EPS)) for g in p["g"]]
    W = [w * si[None, :] for w, si in zip(p["W"], s)]
    b = [bi * si + bt for bi, si, bt in zip(p["b"], s, p["bt"])]
    return W, b


def _tables1_body(x_ref, pos_ref, wx_ref, wp_ref, b_ref, u_ref, v_ref):
    v = jnp.dot(pos_ref[...], wp_ref[...], preferred_element_type=jnp.float32)
    u = jnp.dot(x_ref[...], wx_ref[...], preferred_element_type=jnp.float32)
    u_ref[...] = u + v + b_ref[...]
    v_ref[...] = v


def _pool_tables_body(xe_ref, xo_ref, pe_ref, po_ref, wx_ref, wp_ref, b_ref,
                      u_ref, v_ref, pn_ref):
    xn = jnp.maximum(xe_ref[...], xo_ref[...])
    pn = 0.5 * (pe_ref[...] + po_ref[...])
    v = jnp.dot(pn, wp_ref[...], preferred_element_type=jnp.float32)
    u = jnp.dot(xn, wx_ref[...], preferred_element_type=jnp.float32)
    u_ref[...] = u + v + b_ref[...]
    v_ref[...] = v
    pn_ref[...] = pn


def _mlp2_body(g_ref, w2_ref, b2_ref, w3_ref, b3_ref, o_ref):
    h = jnp.maximum(g_ref[...], 0.0)
    h = jnp.dot(h, w2_ref[...], preferred_element_type=jnp.float32) + b2_ref[...]
    h = jnp.maximum(h, 0.0)
    o_ref[...] = jnp.dot(h, w3_ref[...], preferred_element_type=jnp.float32) + b3_ref[...]


def _head_body(g_ref, w0_ref, b0_ref, w1_ref, b1_ref, w2_ref, b2_ref, o_ref):
    h = jnp.maximum(g_ref[...], 0.0)
    h = jnp.maximum(jnp.dot(h, w0_ref[...], preferred_element_type=jnp.float32) + b0_ref[...], 0.0)
    h = jnp.maximum(jnp.dot(h, w1_ref[...], preferred_element_type=jnp.float32) + b1_ref[...], 0.0)
    o_ref[...] = jnp.dot(h, w2_ref[...], preferred_element_type=jnp.float32) + b2_ref[...]


def _mlp2(g, w2, b2, w3, b3, block):
    e, hin = g.shape
    hmid, hout = w3.shape[0], w3.shape[1]
    assert e % block == 0
    return pl.pallas_call(
        _mlp2_body,
        grid=(e // block,),
        in_specs=[
            pl.BlockSpec((block, hin), lambda i: (i, 0)),
            pl.BlockSpec((hin, hmid), lambda i: (0, 0)),
            pl.BlockSpec((1, hmid), lambda i: (0, 0)),
            pl.BlockSpec((hmid, hout), lambda i: (0, 0)),
            pl.BlockSpec((1, hout), lambda i: (0, 0)),
        ],
        out_specs=pl.BlockSpec((block, hout), lambda i: (i, 0)),
        out_shape=jax.ShapeDtypeStruct((e, hout), jnp.float32),
    )(g, w2, b2[None, :], w3, b3[None, :])


def _conv_tables1(x, pos8, wx, wp, b):
    n, h = x.shape[0], wx.shape[1]
    outs = pl.pallas_call(
        _tables1_body,
        out_shape=(jax.ShapeDtypeStruct((n, h), jnp.float32),
                   jax.ShapeDtypeStruct((n, h), jnp.float32)),
    )(x, pos8, wx, wp, b[None, :])
    return outs


def _pool_tables(xprev, pos8, wx, wp, b):
    n2, hin = xprev.shape
    n = n2 // 2
    h = wx.shape[1]
    return pl.pallas_call(
        _pool_tables_body,
        out_shape=(jax.ShapeDtypeStruct((n, h), jnp.float32),
                   jax.ShapeDtypeStruct((n, h), jnp.float32),
                   jax.ShapeDtypeStruct((n, 8), jnp.float32)),
    )(xprev[0::2], xprev[1::2], pos8[0::2], pos8[1::2], wx, wp, b[None, :])


def kernel(x, pos, edge_index, batch, params):
    src, dst = edge_index[0], edge_index[1]
    n = x.shape[0]
    pos8 = jnp.pad(pos, ((0, 0), (0, 5)))

    W, bb = _fold(params["conv1"])
    wx1, wp1 = W[0][:x.shape[1]], jnp.pad(W[0][x.shape[1]:], ((0, 5), (0, 0)))
    u1, v1 = _conv_tables1(x, pos8, wx1, wp1, bb[0])
    g1 = jnp.concatenate([u1[src] - v1[dst], u1 - v1], axis=0)
    h1 = _mlp2(g1, W[1], bb[1], W[2], bb[2], block=2000)
    dst_ext = jnp.concatenate([dst, jnp.arange(n, dtype=dst.dtype)])
    out1 = jax.ops.segment_max(h1, dst_ext, num_segments=n)

    W, bb = _fold(params["conv2"])
    hid2 = out1.shape[1]
    wx2, wp2 = W[0][:hid2], jnp.pad(W[0][hid2:], ((0, 5), (0, 0)))
    u2, v2, pos8_1 = _pool_tables(out1, pos8, wx2, wp2, bb[0])
    src2, dst2 = src[0::2] // 2, dst[0::2] // 2
    g2 = u2[src2] - v2[dst2]
    h2 = _mlp2(g2, W[1], bb[1], W[2], bb[2], block=2000)
    out2 = jax.ops.segment_max(h2, dst2, num_segments=n // 2)
    out2 = jnp.where(jnp.isfinite(out2), out2, 0.0)

    W, bb = _fold(params["conv3"])
    hid3 = out2.shape[1]
    wx3, wp3 = W[0][:hid3], jnp.pad(W[0][hid3:], ((0, 5), (0, 0)))
    u3, v3, _ = _pool_tables(out2, pos8_1, wx3, wp3, bb[0])
    src3, dst3 = src[0::4] // 4, dst[0::4] // 4
    g3 = u3[src3] - v3[dst3]
    h3 = _mlp2(g3, W[1], bb[1], W[2], bb[2], block=1000)
    out3 = jax.ops.segment_max(h3, dst3, num_segments=n // 4)
    out3 = jnp.where(jnp.isfinite(out3), out3, 0.0)

    b2 = batch[0::4]
    g = jax.ops.segment_max(out3, b2, num_segments=B)
    g = jnp.where(jnp.isfinite(g), g, 0.0)
    hp = params["head"]
    g = hp["g0"] * (1.0 / jnp.sqrt(1.0 + EPS)) * g + hp["bt0"]
    W, bb = _fold(hp)
    out = pl.pallas_call(
        _head_body,
        out_shape=jax.ShapeDtypeStruct((B, NC), jnp.float32),
    )(g, W[0], bb[0][None, :], W[1], bb[1][None, :], W[2], bb[2][None, :])
    return out


# speedup vs baseline: 1.5647x; 1.5647x over previous
import functools

import jax
import jax.numpy as jnp
from jax import lax
from jax.experimental import pallas as pl
from jax.experimental.pallas import tpu as pltpu
from jax.experimental.pallas import tpu_sc as plsc

EPS = 1e-5
B = 8
NC = 40

_NW = 32


def _sc_gather(tab, isrc, idst, h, chunk):
    e_tot = isrc.shape[0]
    assert e_tot % (_NW * chunk) == 0 and chunk % 8 == 0
    e_per_w = e_tot // _NW
    mesh = plsc.VectorSubcoreMesh(core_axis_name="c", subcore_axis_name="s")

    @functools.partial(
        pl.kernel,
        mesh=mesh,
        out_type=(jax.ShapeDtypeStruct((e_tot, h), jnp.float32),
                  jax.ShapeDtypeStruct((e_tot, h), jnp.float32)),
        scratch_types=[
            pltpu.VMEM((chunk,), jnp.int32),
            pltpu.VMEM((chunk, h), jnp.float32),
            pltpu.SemaphoreType.DMA,
        ],
    )
    def k(tab_hbm, isrc_hbm, idst_hbm, gs_hbm, gd_hbm, idx_v, rows_v, sem):
        wid = lax.axis_index("s") * 2 + lax.axis_index("c")
        base = wid * e_per_w
        for j in range(e_per_w // chunk):
            off = base + j * chunk
            pltpu.sync_copy(isrc_hbm.at[pl.ds(off, chunk)], idx_v)
            pltpu.async_copy(tab_hbm.at[idx_v], rows_v, sem).wait()
            pltpu.sync_copy(rows_v, gs_hbm.at[pl.ds(off, chunk)])
            pltpu.sync_copy(idst_hbm.at[pl.ds(off, chunk)], idx_v)
            pltpu.async_copy(tab_hbm.at[idx_v], rows_v, sem).wait()
            pltpu.sync_copy(rows_v, gd_hbm.at[pl.ds(off, chunk)])

    return k(tab, isrc, idst)


def _fold(p):
    s = [g * (1.0 / jnp.sqrt(1.0 + EPS)) for g in p["g"]]
    W = [w * si[None, :] for w, si in zip(p["W"], s)]
    b = [bi * si + bt for bi, si, bt in zip(p["b"], s, p["bt"])]
    return W, b


def _tables1_body(x_ref, pos_ref, wx_ref, wp_ref, b_ref, tab_ref):
    n = x_ref.shape[0]
    v = jnp.dot(pos_ref[...], wp_ref[...], preferred_element_type=jnp.float32)
    u = jnp.dot(x_ref[...], wx_ref[...], preferred_element_type=jnp.float32)
    tab_ref[0:n, :] = u + v + b_ref[...]
    tab_ref[n:2 * n, :] = v


def _pool_tables_body(xe_ref, xo_ref, pe_ref, po_ref, wx_ref, wp_ref, b_ref,
                      tab_ref, pn_ref):
    n = xe_ref.shape[0]
    xn = jnp.maximum(xe_ref[...], xo_ref[...])
    pn = 0.5 * (pe_ref[...] + po_ref[...])
    v = jnp.dot(pn, wp_ref[...], preferred_element_type=jnp.float32)
    u = jnp.dot(xn, wx_ref[...], preferred_element_type=jnp.float32)
    tab_ref[0:n, :] = u + v + b_ref[...]
    tab_ref[n:2 * n, :] = v
    pn_ref[...] = pn


def _mlp2_body(gs_ref, gd_ref, w2_ref, b2_ref, w3_ref, b3_ref, o_ref):
    h = jnp.maximum(gs_ref[...] - gd_ref[...], 0.0)
    h = jnp.dot(h, w2_ref[...], preferred_element_type=jnp.float32) + b2_ref[...]
    h = jnp.maximum(h, 0.0)
    o_ref[...] = jnp.dot(h, w3_ref[...], preferred_element_type=jnp.float32) + b3_ref[...]


def _head_body(g_ref, w0_ref, b0_ref, w1_ref, b1_ref, w2_ref, b2_ref, o_ref):
    h = jnp.maximum(g_ref[...], 0.0)
    h = jnp.maximum(jnp.dot(h, w0_ref[...], preferred_element_type=jnp.float32) + b0_ref[...], 0.0)
    h = jnp.maximum(jnp.dot(h, w1_ref[...], preferred_element_type=jnp.float32) + b1_ref[...], 0.0)
    o_ref[...] = jnp.dot(h, w2_ref[...], preferred_element_type=jnp.float32) + b2_ref[...]


def _mlp2(gs, gd, w2, b2, w3, b3, block):
    e, hin = gs.shape
    hmid, hout = w3.shape[0], w3.shape[1]
    assert e % block == 0
    return pl.pallas_call(
        _mlp2_body,
        grid=(e // block,),
        in_specs=[
            pl.BlockSpec((block, hin), lambda i: (i, 0)),
            pl.BlockSpec((block, hin), lambda i: (i, 0)),
            pl.BlockSpec((hin, hmid), lambda i: (0, 0)),
            pl.BlockSpec((1, hmid), lambda i: (0, 0)),
            pl.BlockSpec((hmid, hout), lambda i: (0, 0)),
            pl.BlockSpec((1, hout), lambda i: (0, 0)),
        ],
        out_specs=pl.BlockSpec((block, hout), lambda i: (i, 0)),
        out_shape=jax.ShapeDtypeStruct((e, hout), jnp.float32),
    )(gs, gd, w2, b2[None, :], w3, b3[None, :])


def _conv_tables1(x, pos8, wx, wp, b):
    n, h = x.shape[0], wx.shape[1]
    return pl.pallas_call(
        _tables1_body,
        out_shape=jax.ShapeDtypeStruct((2 * n, h), jnp.float32),
    )(x, pos8, wx, wp, b[None, :])


def _pool_tables(xprev, pos8, wx, wp, b):
    n2, hin = xprev.shape
    n = n2 // 2
    h = wx.shape[1]
    return pl.pallas_call(
        _pool_tables_body,
        out_shape=(jax.ShapeDtypeStruct((2 * n, h), jnp.float32),
                   jax.ShapeDtypeStruct((n, 8), jnp.float32)),
    )(xprev[0::2], xprev[1::2], pos8[0::2], pos8[1::2], wx, wp, b[None, :])


def kernel(x, pos, edge_index, batch, params):
    src, dst = edge_index[0], edge_index[1]
    n = x.shape[0]
    e = src.shape[0]
    pos8 = jnp.pad(pos, ((0, 0), (0, 5)))

    W, bb = _fold(params["conv1"])
    wx1 = jnp.pad(W[0][:x.shape[1]], ((0, 0), (0, 64)))
    wp1 = jnp.pad(W[0][x.shape[1]:], ((0, 5), (0, 64)))
    b1 = jnp.pad(bb[0], (0, 64))
    w2_1 = jnp.pad(W[1], ((0, 64), (0, 0)))
    tab1 = _conv_tables1(x, pos8, wx1, wp1, b1)
    gs1, gd1 = _sc_gather(tab1, src, dst + n, 128, chunk=200)
    h1e = _mlp2(gs1, gd1, w2_1, bb[1], W[2], bb[2], block=2000)
    h1s = _mlp2(tab1[:n], tab1[n:], w2_1, bb[1], W[2], bb[2], block=2000)
    out1 = jnp.maximum(jax.ops.segment_max(h1e, dst, num_segments=n), h1s)

    W, bb = _fold(params["conv2"])
    hid2 = out1.shape[1]
    n2 = n // 2
    e2p = 81920
    wx2, wp2 = W[0][:hid2], jnp.pad(W[0][hid2:], ((0, 5), (0, 0)))
    tab2, pos8_1 = _pool_tables(out1, pos8, wx2, wp2, bb[0])
    src2 = jnp.pad(src[0::2] // 2, (0, e2p - e // 2))
    dst2 = dst[0::2] // 2
    idst2 = jnp.pad(dst2 + n2, (0, e2p - e // 2))
    gs2, gd2 = _sc_gather(tab2, src2, idst2, 128, chunk=512)
    h2 = _mlp2(gs2, gd2, W[1], bb[1], W[2], bb[2], block=2048)
    dst2p = jnp.pad(dst2, (0, e2p - e // 2), constant_values=n2)
    out2 = jax.ops.segment_max(h2, dst2p, num_segments=n2)
    out2 = jnp.where(jnp.isfinite(out2), out2, 0.0)

    W, bb = _fold(params["conv3"])
    hid3 = out2.shape[1]
    n3 = n // 4
    e3p = 40960
    wx3, wp3 = W[0][:hid3], jnp.pad(W[0][hid3:], ((0, 5), (0, 0)))
    tab3, _ = _pool_tables(out2, pos8_1, wx3, wp3, bb[0])
    src3 = jnp.pad(src[0::4] // 4, (0, e3p - e // 4))
    dst3 = dst[0::4] // 4
    idst3 = jnp.pad(dst3 + n3, (0, e3p - e // 4))
    gs3, gd3 = _sc_gather(tab3, src3, idst3, 256, chunk=256)
    h3 = _mlp2(gs3, gd3, W[1], bb[1], W[2], bb[2], block=1024)
    dst3p = jnp.pad(dst3, (0, e3p - e // 4), constant_values=n3)
    out3 = jax.ops.segment_max(h3, dst3p, num_segments=n3)
    out3 = jnp.where(jnp.isfinite(out3), out3, 0.0)

    b2 = batch[0::4]
    g = jax.ops.segment_max(out3, b2, num_segments=B)
    g = jnp.where(jnp.isfinite(g), g, 0.0)
    hp = params["head"]
    g = hp["g0"] * (1.0 / jnp.sqrt(1.0 + EPS)) * g + hp["bt0"]
    W, bb = _fold(hp)
    out = pl.pallas_call(
        _head_body,
        out_shape=jax.ShapeDtypeStruct((B, NC), jnp.float32),
    )(g, W[0], bb[0][None, :], W[1], bb[1][None, :], W[2], bb[2][None, :])
    return out
